# baseline (speedup 1.0000x reference)
import jax
import jax.numpy as jnp
from jax import lax
from jax.experimental import pallas as pl
from jax.experimental.pallas import tpu as pltpu
from jax.experimental.pallas import tpu_sc as plsc

N = 10000
D = 128
E = 320000
NC = 2
NS = 16
CHUNK = 80
EDGES_PER_TILE = E // (NC * NS)
PHASES = 5
EPP = EDGES_PER_TILE // PHASES
NCHP = EPP // CHUNK
STRIPE = 624
LANES = 16


def _sc_body(dst_hbm, src_hbm, an_hbm, x_hbm, out_hbm,
             acc, src_v, dst_v, an_v, dst_c0, dst_c1,
             g0, g1, sb0, sb1, gsem0, gsem1, ssem0, ssem1, isem):
    c = lax.axis_index("c")
    s = lax.axis_index("s")
    tile = c * NS + s

    gbufs = (g0, g1)
    sbufs = (sb0, sb1)
    gsems = (gsem0, gsem1)
    ssems = (ssem0, ssem1)
    dstc = (dst_c0, dst_c1)

    def _prefetch(ph):
        eb = tile * EDGES_PER_TILE + ph * EPP
        pltpu.async_copy(src_hbm.at[pl.ds(eb, EPP)], src_v, isem)
        pltpu.async_copy(dst_hbm.at[pl.ds(eb, EPP)], dst_v, isem)
        pltpu.async_copy(an_hbm.at[pl.ds(eb, EPP)], an_v, isem)

    def _pwait(ph):
        eb = tile * EDGES_PER_TILE + ph * EPP
        pltpu.make_async_copy(src_hbm.at[pl.ds(eb, EPP)], src_v, isem).wait()
        pltpu.make_async_copy(dst_hbm.at[pl.ds(eb, EPP)], dst_v, isem).wait()
        pltpu.make_async_copy(an_hbm.at[pl.ds(eb, EPP)], an_v, isem).wait()

    _prefetch(0)

    def _zrow(r, carry):
        for j in range(D // LANES):
            sb0[r, pl.ds(j * LANES, LANES)] = jnp.zeros((LANES,), jnp.float32)
        return carry
    lax.fori_loop(0, CHUNK, _zrow, 0)
    row0 = s * STRIPE
    for k in range(STRIPE // CHUNK):
        pltpu.async_copy(sb0, acc.at[pl.ds(row0 + k * CHUNK, CHUNK)], gsem0)
    pltpu.async_copy(sb0.at[pl.ds(0, STRIPE % CHUNK)],
                     acc.at[pl.ds(row0 + STRIPE - STRIPE % CHUNK,
                                  STRIPE % CHUNK)], gsem1)

    @pl.when(s == NS - 1)
    def _():
        pltpu.sync_copy(sb0.at[pl.ds(0, N - NS * STRIPE)],
                        acc.at[pl.ds(NS * STRIPE, N - NS * STRIPE)])
    for k in range(STRIPE // CHUNK):
        pltpu.make_async_copy(
            sb0, acc.at[pl.ds(row0 + k * CHUNK, CHUNK)], gsem0).wait()
    pltpu.make_async_copy(
        sb0.at[pl.ds(0, STRIPE % CHUNK)],
        acc.at[pl.ds(row0 + STRIPE - STRIPE % CHUNK, STRIPE % CHUNK)],
        gsem1).wait()
    plsc.subcore_barrier()

    def _run_phase():
        def _gather(j, g):
            pltpu.async_copy(x_hbm.at[src_v.at[pl.ds(j * CHUNK, CHUNK)]],
                             gbufs[g], gsems[g])

        def _gwait(j, g):
            pltpu.make_async_copy(x_hbm.at[src_v.at[pl.ds(j * CHUNK, CHUNK)]],
                                  gbufs[g], gsems[g]).wait()

        def _swait(b):
            pltpu.make_async_copy(sbufs[b], acc.at[dstc[b]], ssems[b]).wait()

        def _process(j, g, b):
            for k in range(CHUNK // LANES):
                dstc[b][pl.ds(k * LANES, LANES)] = (
                    dst_v[pl.ds(j * CHUNK + k * LANES, LANES)])
            gbuf, sbuf = gbufs[g], sbufs[b]

            def _grp(grp, cc):
                e0 = grp * LANES
                an16 = an_v[pl.ds(j * CHUNK + e0, LANES)]
                for l in range(LANES):
                    w = an16[l]
                    for jj in range(D // LANES):
                        sl = pl.ds(jj * LANES, LANES)
                        sbuf[e0 + l, sl] = gbuf[e0 + l, sl] * w
                return cc
            lax.fori_loop(0, CHUNK // LANES, _grp, 0)
            pltpu.async_copy(sbuf, acc.at[dstc[b]], ssems[b], add=True)

        _gather(0, 0)
        _gather(1, 1)

        def _duo(q, cc):
            _gwait(2 * q, 0)

            @pl.when(q > 0)
            def _():
                _swait(0)
            _process(2 * q, 0, 0)
            _gather(2 * q + 2, 0)

            _gwait(2 * q + 1, 1)

            @pl.when(q > 0)
            def _():
                _swait(1)
            _process(2 * q + 1, 1, 1)

            @pl.when(q < NCHP // 2 - 1)
            def _():
                _gather(2 * q + 3, 1)
            return cc
        lax.fori_loop(0, NCHP // 2, _duo, 0)
        _gwait(NCHP - 1, 0)
        _swait(0)
        _process(NCHP - 1, 0, 0)
        _swait(1)
        _swait(0)

    def _phase(ph, carry):
        _pwait(ph)
        _run_phase()

        @pl.when(ph < PHASES - 1)
        def _():
            _prefetch(ph + 1)
        return carry
    lax.fori_loop(0, PHASES, _phase, 0)
    plsc.subcore_barrier()

    pltpu.sync_copy(acc.at[pl.ds(row0, STRIPE)],
                    out_hbm.at[pl.ds(c * N + row0, STRIPE)])

    @pl.when(s == NS - 1)
    def _():
        pltpu.sync_copy(acc.at[pl.ds(NS * STRIPE, N - NS * STRIPE)],
                        out_hbm.at[pl.ds(c * N + NS * STRIPE,
                                         N - NS * STRIPE)])


@jax.jit
def _sc_spmm(dst, src, an_values, x):
    mesh = plsc.VectorSubcoreMesh(core_axis_name="c", subcore_axis_name="s")
    return pl.kernel(
        _sc_body,
        out_type=jax.ShapeDtypeStruct((NC * N, D), jnp.float32),
        mesh=mesh,
        scratch_types=[
            pltpu.VMEM_SHARED((N, D), jnp.float32),
            pltpu.VMEM((EPP,), jnp.int32),
            pltpu.VMEM((EPP,), jnp.int32),
            pltpu.VMEM((EPP,), jnp.float32),
            pltpu.VMEM((CHUNK,), jnp.int32),
            pltpu.VMEM((CHUNK,), jnp.int32),
            pltpu.VMEM((CHUNK, D), jnp.float32),
            pltpu.VMEM((CHUNK, D), jnp.float32),
            pltpu.VMEM((CHUNK, D), jnp.float32),
            pltpu.VMEM((CHUNK, D), jnp.float32),
            pltpu.SemaphoreType.DMA,
            pltpu.SemaphoreType.DMA,
            pltpu.SemaphoreType.DMA,
            pltpu.SemaphoreType.DMA,
            pltpu.SemaphoreType.DMA,
        ],
    )(dst, src, an_values, x)


BLK = 1000


def _mm_body(a_ref, b_ref, w_ref, bias_ref, o_ref):
    o_ref[...] = jnp.dot(a_ref[0] + b_ref[0], w_ref[...],
                         preferred_element_type=jnp.float32) + bias_ref[...]


@jax.jit
def _combine_matmul(partials, weight, bias2d):
    grid = (N // BLK,)
    return pl.pallas_call(
        _mm_body,
        grid=grid,
        in_specs=[
            pl.BlockSpec((1, BLK, D), lambda i: (0, i, 0)),
            pl.BlockSpec((1, BLK, D), lambda i: (1, i, 0)),
            pl.BlockSpec((D, D), lambda i: (0, 0)),
            pl.BlockSpec((1, D), lambda i: (0, 0)),
        ],
        out_specs=pl.BlockSpec((BLK, D), lambda i: (i, 0)),
        out_shape=jax.ShapeDtypeStruct((N, D), jnp.float32),
    )(partials, partials, weight, bias2d)


def kernel(x, edge_index, an_values, weight, bias):
    dst = edge_index[0]
    src = edge_index[1]
    partials = _sc_spmm(dst, src, an_values, x).reshape(NC, N, D)
    return _combine_matmul(partials, weight, bias.reshape(1, D))


# speedup vs baseline: 1.1166x; 1.1166x over previous
import jax
import jax.numpy as jnp
from jax import lax
from jax.experimental import pallas as pl
from jax.experimental.pallas import tpu as pltpu
from jax.experimental.pallas import tpu_sc as plsc

N = 10000
D = 128
E = 320000
NC = 2
NS = 16
CHUNK = 80
EDGES_PER_TILE = E // (NC * NS)
PHASES = 5
EPP = EDGES_PER_TILE // PHASES
NCHP = EPP // CHUNK
STRIPE = 624
LANES = 16
NBUF = 3


def _sc_body(dst_hbm, src_hbm, an_hbm, x_hbm, out_hbm,
             acc, srcA, dstA, anA, srcB, dstB, anB,
             dst_c0, dst_c1, dst_c2, rows0, rows1, rows2,
             gsem0, gsem1, gsem2, ssem0, ssem1, ssem2, isemA, isemB):
    c = lax.axis_index("c")
    s = lax.axis_index("s")
    tile = c * NS + s

    bufs = (rows0, rows1, rows2)
    gsems = (gsem0, gsem1, gsem2)
    ssems = (ssem0, ssem1, ssem2)
    dstc = (dst_c0, dst_c1, dst_c2)
    setA = (srcA, dstA, anA)
    setB = (srcB, dstB, anB)

    def _prefetch(ph, st, isem):
        eb = tile * EDGES_PER_TILE + ph * EPP
        pltpu.async_copy(src_hbm.at[pl.ds(eb, EPP)], st[0], isem)
        pltpu.async_copy(dst_hbm.at[pl.ds(eb, EPP)], st[1], isem)
        pltpu.async_copy(an_hbm.at[pl.ds(eb, EPP)], st[2], isem)

    def _pwait(ph, st, isem):
        eb = tile * EDGES_PER_TILE + ph * EPP
        pltpu.make_async_copy(src_hbm.at[pl.ds(eb, EPP)], st[0], isem).wait()
        pltpu.make_async_copy(dst_hbm.at[pl.ds(eb, EPP)], st[1], isem).wait()
        pltpu.make_async_copy(an_hbm.at[pl.ds(eb, EPP)], st[2], isem).wait()

    _prefetch(0, setA, isemA)
    _prefetch(1, setB, isemB)

    def _zrow(r, carry):
        for j in range(D // LANES):
            rows0[r, pl.ds(j * LANES, LANES)] = jnp.zeros((LANES,), jnp.float32)
        return carry
    lax.fori_loop(0, CHUNK, _zrow, 0)
    row0 = s * STRIPE
    for k in range(STRIPE // CHUNK):
        pltpu.async_copy(rows0, acc.at[pl.ds(row0 + k * CHUNK, CHUNK)], gsem0)
    pltpu.async_copy(rows0.at[pl.ds(0, STRIPE % CHUNK)],
                     acc.at[pl.ds(row0 + STRIPE - STRIPE % CHUNK,
                                  STRIPE % CHUNK)], gsem1)

    @pl.when(s == NS - 1)
    def _():
        pltpu.sync_copy(rows0.at[pl.ds(0, N - NS * STRIPE)],
                        acc.at[pl.ds(NS * STRIPE, N - NS * STRIPE)])
    for k in range(STRIPE // CHUNK):
        pltpu.make_async_copy(
            rows0, acc.at[pl.ds(row0 + k * CHUNK, CHUNK)], gsem0).wait()
    pltpu.make_async_copy(
        rows0.at[pl.ds(0, STRIPE % CHUNK)],
        acc.at[pl.ds(row0 + STRIPE - STRIPE % CHUNK, STRIPE % CHUNK)],
        gsem1).wait()
    plsc.subcore_barrier()

    def _run_phase(sv, dv, av):
        def _gather(j, b):
            pltpu.async_copy(x_hbm.at[sv.at[pl.ds(j * CHUNK, CHUNK)]],
                             bufs[b], gsems[b])

        def _gwait(j, b):
            pltpu.make_async_copy(x_hbm.at[sv.at[pl.ds(j * CHUNK, CHUNK)]],
                                  bufs[b], gsems[b]).wait()

        def _swait(b):
            pltpu.make_async_copy(bufs[b], acc.at[dstc[b]], ssems[b]).wait()

        def _process(j, b):
            for k in range(CHUNK // LANES):
                dstc[b][pl.ds(k * LANES, LANES)] = (
                    dv[pl.ds(j * CHUNK + k * LANES, LANES)])
            buf = bufs[b]

            def _grp(g, cc):
                e0 = g * LANES
                an16 = av[pl.ds(j * CHUNK + e0, LANES)]
                for l in range(LANES):
                    w = an16[l]
                    for jj in range(D // LANES):
                        sl = pl.ds(jj * LANES, LANES)
                        buf[e0 + l, sl] = buf[e0 + l, sl] * w
                return cc
            lax.fori_loop(0, CHUNK // LANES, _grp, 0)
            pltpu.async_copy(buf, acc.at[dstc[b]], ssems[b], add=True)

        _gather(0, 0)
        _gather(1, 1)

        def _trip(q, cc):
            @pl.when(q > 0)
            def _():
                _swait(2)
            _gather(3 * q + 2, 2)
            _gwait(3 * q, 0)
            _process(3 * q, 0)

            _swait(0)
            _gather(3 * q + 3, 0)
            _gwait(3 * q + 1, 1)
            _process(3 * q + 1, 1)

            _swait(1)

            @pl.when(q < NCHP // 3 - 1)
            def _():
                _gather(3 * q + 4, 1)
            _gwait(3 * q + 2, 2)
            _process(3 * q + 2, 2)
            return cc
        lax.fori_loop(0, NCHP // 3, _trip, 0)
        _swait(2)
        _gwait(NCHP - 1, 0)
        _process(NCHP - 1, 0)
        _swait(0)

    def _pair(pp, carry):
        ph0 = 2 * pp
        _pwait(ph0, setA, isemA)
        _run_phase(*setA)
        _prefetch(ph0 + 2, setA, isemA)
        _pwait(ph0 + 1, setB, isemB)
        _run_phase(*setB)

        @pl.when(pp == 0)
        def _():
            _prefetch(ph0 + 3, setB, isemB)
        return carry
    lax.fori_loop(0, 2, _pair, 0)
    _pwait(PHASES - 1, setA, isemA)
    _run_phase(*setA)
    plsc.subcore_barrier()

    pltpu.sync_copy(acc.at[pl.ds(row0, STRIPE)],
                    out_hbm.at[pl.ds(c * N + row0, STRIPE)])

    @pl.when(s == NS - 1)
    def _():
        pltpu.sync_copy(acc.at[pl.ds(NS * STRIPE, N - NS * STRIPE)],
                        out_hbm.at[pl.ds(c * N + NS * STRIPE,
                                         N - NS * STRIPE)])


@jax.jit
def _sc_spmm(dst, src, an_values, x):
    mesh = plsc.VectorSubcoreMesh(core_axis_name="c", subcore_axis_name="s")
    return pl.kernel(
        _sc_body,
        out_type=jax.ShapeDtypeStruct((NC * N, D), jnp.float32),
        mesh=mesh,
        scratch_types=[
            pltpu.VMEM_SHARED((N, D), jnp.float32),
            pltpu.VMEM((EPP,), jnp.int32),
            pltpu.VMEM((EPP,), jnp.int32),
            pltpu.VMEM((EPP,), jnp.float32),
            pltpu.VMEM((EPP,), jnp.int32),
            pltpu.VMEM((EPP,), jnp.int32),
            pltpu.VMEM((EPP,), jnp.float32),
            pltpu.VMEM((CHUNK,), jnp.int32),
            pltpu.VMEM((CHUNK,), jnp.int32),
            pltpu.VMEM((CHUNK,), jnp.int32),
            pltpu.VMEM((CHUNK, D), jnp.float32),
            pltpu.VMEM((CHUNK, D), jnp.float32),
            pltpu.VMEM((CHUNK, D), jnp.float32),
            pltpu.SemaphoreType.DMA,
            pltpu.SemaphoreType.DMA,
            pltpu.SemaphoreType.DMA,
            pltpu.SemaphoreType.DMA,
            pltpu.SemaphoreType.DMA,
            pltpu.SemaphoreType.DMA,
            pltpu.SemaphoreType.DMA,
            pltpu.SemaphoreType.DMA,
        ],
    )(dst, src, an_values, x)


BLK = 1000


def _mm_body(a_ref, b_ref, w_ref, bias_ref, o_ref):
    o_ref[...] = jnp.dot(a_ref[0] + b_ref[0], w_ref[...],
                         preferred_element_type=jnp.float32) + bias_ref[...]


@jax.jit
def _combine_matmul(partials, weight, bias2d):
    grid = (N // BLK,)
    return pl.pallas_call(
        _mm_body,
        grid=grid,
        in_specs=[
            pl.BlockSpec((1, BLK, D), lambda i: (0, i, 0)),
            pl.BlockSpec((1, BLK, D), lambda i: (1, i, 0)),
            pl.BlockSpec((D, D), lambda i: (0, 0)),
            pl.BlockSpec((1, D), lambda i: (0, 0)),
        ],
        out_specs=pl.BlockSpec((BLK, D), lambda i: (i, 0)),
        out_shape=jax.ShapeDtypeStruct((N, D), jnp.float32),
    )(partials, partials, weight, bias2d)


def kernel(x, edge_index, an_values, weight, bias):
    dst = edge_index[0]
    src = edge_index[1]
    partials = _sc_spmm(dst, src, an_values, x).reshape(NC, N, D)
    return _combine_matmul(partials, weight, bias.reshape(1, D))


# speedup vs baseline: 1.1715x; 1.0492x over previous
import jax
import jax.numpy as jnp
from jax import lax
from jax.experimental import pallas as pl
from jax.experimental.pallas import tpu as pltpu
from jax.experimental.pallas import tpu_sc as plsc

N = 10000
D = 128
E = 320000
NC = 2
NS = 16
CHUNK = 80
EDGES_PER_TILE = E // (NC * NS)
PHASES = 5
EPP = EDGES_PER_TILE // PHASES
NCHP = EPP // CHUNK
STRIPE = 624
LANES = 16
NBUF = 3


def _sc_body(edge_hbm, an_hbm, x_hbm, out_hbm,
             acc, srcA, dstA, anA, srcB, dstB, anB,
             dst_c0, dst_c1, dst_c2, rows0, rows1, rows2,
             gsem0, gsem1, gsem2, ssem0, ssem1, ssem2, isemA, isemB):
    c = lax.axis_index("c")
    s = lax.axis_index("s")
    tile = c * NS + s

    bufs = (rows0, rows1, rows2)
    gsems = (gsem0, gsem1, gsem2)
    ssems = (ssem0, ssem1, ssem2)
    dstc = (dst_c0, dst_c1, dst_c2)
    setA = (srcA, dstA, anA)
    setB = (srcB, dstB, anB)

    def _prefetch(ph, st, isem):
        eb = tile * EDGES_PER_TILE + ph * EPP
        pltpu.async_copy(edge_hbm.at[pl.ds(E + eb, EPP)], st[0], isem)
        pltpu.async_copy(edge_hbm.at[pl.ds(eb, EPP)], st[1], isem)
        pltpu.async_copy(an_hbm.at[pl.ds(eb, EPP)], st[2], isem)

    def _pwait(ph, st, isem):
        eb = tile * EDGES_PER_TILE + ph * EPP
        pltpu.make_async_copy(edge_hbm.at[pl.ds(E + eb, EPP)],
                              st[0], isem).wait()
        pltpu.make_async_copy(edge_hbm.at[pl.ds(eb, EPP)], st[1], isem).wait()
        pltpu.make_async_copy(an_hbm.at[pl.ds(eb, EPP)], st[2], isem).wait()

    _prefetch(0, setA, isemA)
    _prefetch(1, setB, isemB)

    def _zrow(r, carry):
        for j in range(D // LANES):
            rows0[r, pl.ds(j * LANES, LANES)] = jnp.zeros((LANES,), jnp.float32)
        return carry
    lax.fori_loop(0, CHUNK, _zrow, 0)
    row0 = s * STRIPE
    for k in range(STRIPE // CHUNK):
        pltpu.async_copy(rows0, acc.at[pl.ds(row0 + k * CHUNK, CHUNK)], gsem0)
    pltpu.async_copy(rows0.at[pl.ds(0, STRIPE % CHUNK)],
                     acc.at[pl.ds(row0 + STRIPE - STRIPE % CHUNK,
                                  STRIPE % CHUNK)], gsem1)

    @pl.when(s == NS - 1)
    def _():
        pltpu.sync_copy(rows0.at[pl.ds(0, N - NS * STRIPE)],
                        acc.at[pl.ds(NS * STRIPE, N - NS * STRIPE)])
    for k in range(STRIPE // CHUNK):
        pltpu.make_async_copy(
            rows0, acc.at[pl.ds(row0 + k * CHUNK, CHUNK)], gsem0).wait()
    pltpu.make_async_copy(
        rows0.at[pl.ds(0, STRIPE % CHUNK)],
        acc.at[pl.ds(row0 + STRIPE - STRIPE % CHUNK, STRIPE % CHUNK)],
        gsem1).wait()
    plsc.subcore_barrier()

    def _run_phase(sv, dv, av):
        def _gather(j, b):
            pltpu.async_copy(x_hbm.at[sv.at[pl.ds(j * CHUNK, CHUNK)]],
                             bufs[b], gsems[b])

        def _gwait(j, b):
            pltpu.make_async_copy(x_hbm.at[sv.at[pl.ds(j * CHUNK, CHUNK)]],
                                  bufs[b], gsems[b]).wait()

        def _swait(b):
            pltpu.make_async_copy(bufs[b], acc.at[dstc[b]], ssems[b]).wait()

        def _process(j, b):
            for k in range(CHUNK // LANES):
                dstc[b][pl.ds(k * LANES, LANES)] = (
                    dv[pl.ds(j * CHUNK + k * LANES, LANES)])
            buf = bufs[b]

            def _grp(g, cc):
                e0 = g * LANES
                an16 = av[pl.ds(j * CHUNK + e0, LANES)]
                for l in range(LANES):
                    w = an16[l]
                    for jj in range(D // LANES):
                        sl = pl.ds(jj * LANES, LANES)
                        buf[e0 + l, sl] = buf[e0 + l, sl] * w
                return cc
            lax.fori_loop(0, CHUNK // LANES, _grp, 0)
            pltpu.async_copy(buf, acc.at[dstc[b]], ssems[b], add=True)

        _gather(0, 0)
        _gather(1, 1)

        def _trip(q, cc):
            @pl.when(q > 0)
            def _():
                _swait(2)
            _gather(3 * q + 2, 2)
            _gwait(3 * q, 0)
            _process(3 * q, 0)

            _swait(0)
            _gather(3 * q + 3, 0)
            _gwait(3 * q + 1, 1)
            _process(3 * q + 1, 1)

            _swait(1)

            @pl.when(q < NCHP // 3 - 1)
            def _():
                _gather(3 * q + 4, 1)
            _gwait(3 * q + 2, 2)
            _process(3 * q + 2, 2)
            return cc
        lax.fori_loop(0, NCHP // 3, _trip, 0)
        _swait(2)
        _gwait(NCHP - 1, 0)
        _process(NCHP - 1, 0)
        _swait(0)

    def _pair(pp, carry):
        ph0 = 2 * pp
        _pwait(ph0, setA, isemA)
        _run_phase(*setA)
        _prefetch(ph0 + 2, setA, isemA)
        _pwait(ph0 + 1, setB, isemB)
        _run_phase(*setB)

        @pl.when(pp == 0)
        def _():
            _prefetch(ph0 + 3, setB, isemB)
        return carry
    lax.fori_loop(0, 2, _pair, 0)
    _pwait(PHASES - 1, setA, isemA)
    _run_phase(*setA)
    plsc.subcore_barrier()

    pltpu.sync_copy(acc.at[pl.ds(row0, STRIPE)],
                    out_hbm.at[pl.ds(c * N + row0, STRIPE)])

    @pl.when(s == NS - 1)
    def _():
        pltpu.sync_copy(acc.at[pl.ds(NS * STRIPE, N - NS * STRIPE)],
                        out_hbm.at[pl.ds(c * N + NS * STRIPE,
                                         N - NS * STRIPE)])


@jax.jit
def _sc_spmm(edge_flat, an_values, x):
    mesh = plsc.VectorSubcoreMesh(core_axis_name="c", subcore_axis_name="s")
    return pl.kernel(
        _sc_body,
        out_type=jax.ShapeDtypeStruct((NC * N, D), jnp.float32),
        mesh=mesh,
        scratch_types=[
            pltpu.VMEM_SHARED((N, D), jnp.float32),
            pltpu.VMEM((EPP,), jnp.int32),
            pltpu.VMEM((EPP,), jnp.int32),
            pltpu.VMEM((EPP,), jnp.float32),
            pltpu.VMEM((EPP,), jnp.int32),
            pltpu.VMEM((EPP,), jnp.int32),
            pltpu.VMEM((EPP,), jnp.float32),
            pltpu.VMEM((CHUNK,), jnp.int32),
            pltpu.VMEM((CHUNK,), jnp.int32),
            pltpu.VMEM((CHUNK,), jnp.int32),
            pltpu.VMEM((CHUNK, D), jnp.float32),
            pltpu.VMEM((CHUNK, D), jnp.float32),
            pltpu.VMEM((CHUNK, D), jnp.float32),
            pltpu.SemaphoreType.DMA,
            pltpu.SemaphoreType.DMA,
            pltpu.SemaphoreType.DMA,
            pltpu.SemaphoreType.DMA,
            pltpu.SemaphoreType.DMA,
            pltpu.SemaphoreType.DMA,
            pltpu.SemaphoreType.DMA,
            pltpu.SemaphoreType.DMA,
        ],
    )(edge_flat, an_values, x)


BLK = 1000


def _mm_body(a_ref, b_ref, w_ref, bias_ref, o_ref):
    o_ref[...] = jnp.dot(a_ref[0] + b_ref[0], w_ref[...],
                         preferred_element_type=jnp.float32) + bias_ref[...]


@jax.jit
def _combine_matmul(partials, weight, bias2d):
    grid = (N // BLK,)
    return pl.pallas_call(
        _mm_body,
        grid=grid,
        in_specs=[
            pl.BlockSpec((1, BLK, D), lambda i: (0, i, 0)),
            pl.BlockSpec((1, BLK, D), lambda i: (1, i, 0)),
            pl.BlockSpec((D, D), lambda i: (0, 0)),
            pl.BlockSpec((1, D), lambda i: (0, 0)),
        ],
        out_specs=pl.BlockSpec((BLK, D), lambda i: (i, 0)),
        out_shape=jax.ShapeDtypeStruct((N, D), jnp.float32),
    )(partials, partials, weight, bias2d)


def kernel(x, edge_index, an_values, weight, bias):
    partials = _sc_spmm(edge_index.reshape(-1), an_values, x).reshape(NC, N, D)
    return _combine_matmul(partials, weight, bias.reshape(1, D))
